# baseline (speedup 1.0000x reference)
import functools

import jax
import jax.numpy as jnp
from jax import lax
from jax.experimental import pallas as pl
from jax.experimental.pallas import tpu as pltpu
from jax.experimental.pallas import tpu_sc as plsc

NODE = 100000
EMB = 128
BATCH = 16384
NC = 2
NS = 16
NW = NC * NS
PER_W = BATCH // NW
CHUNK = 32
N_CHUNK = PER_W // CHUNK
LANES = 16

_mesh = plsc.VectorSubcoreMesh(core_axis_name="c", subcore_axis_name="s")


@functools.partial(
    pl.kernel,
    mesh=_mesh,
    out_type=jax.ShapeDtypeStruct((BATCH,), jnp.float32),
    scratch_types=[
        pltpu.VMEM((PER_W,), jnp.int32),
        pltpu.VMEM((PER_W,), jnp.int32),
        pltpu.VMEM((CHUNK, EMB), jnp.float32),
        pltpu.VMEM((CHUNK, EMB), jnp.float32),
        pltpu.VMEM((CHUNK, EMB), jnp.float32),
        pltpu.VMEM((CHUNK, EMB), jnp.float32),
        pltpu.VMEM((CHUNK, EMB), jnp.float32),
        pltpu.VMEM((CHUNK, EMB), jnp.float32),
        pltpu.VMEM((PER_W,), jnp.float32),
        pltpu.SemaphoreType.DMA,
        pltpu.SemaphoreType.DMA,
        pltpu.SemaphoreType.DMA,
        pltpu.SemaphoreType.DMA,
        pltpu.SemaphoreType.DMA,
        pltpu.SemaphoreType.DMA,
    ],
)
def _sc_dot(emb_hbm, ctx_hbm, ui_hbm, uj_hbm, out_hbm,
            idx_i, idx_j,
            re0, re1, re2, rc0, rc1, rc2, out_v,
            se0, se1, se2, sc0, sc1, sc2):
    c = lax.axis_index("c")
    s = lax.axis_index("s")
    wid = s * NC + c
    base = pl.multiple_of(wid * PER_W, PER_W)
    lane = lax.iota(jnp.int32, LANES)
    bitmask = [((lane >> k) & 1) == 0 for k in range(4)]
    rows_e = (re0, re1, re2)
    rows_c = (rc0, rc1, rc2)
    sem_e = (se0, se1, se2)
    sem_c = (sc0, sc1, sc2)

    pltpu.sync_copy(ui_hbm.at[pl.ds(base, PER_W)], idx_i)
    pltpu.sync_copy(uj_hbm.at[pl.ds(base, PER_W)], idx_j)

    def _issue(ci):
        b = ci % 3
        sl = pl.ds(ci * CHUNK, CHUNK)
        he = pltpu.async_copy(emb_hbm.at[idx_i.at[sl]], rows_e[b], sem_e[b])
        hc = pltpu.async_copy(ctx_hbm.at[idx_j.at[sl]], rows_c[b], sem_c[b])
        return he, hc

    def _perm(x, m):
        return x.at[lane ^ m].get(mode="promise_in_bounds")

    pend = [None, None, None]
    pend[0] = _issue(0)
    pend[1] = _issue(1)
    for ci in range(N_CHUNK):
        if ci + 2 < N_CHUNK:
            pend[(ci + 2) % 3] = _issue(ci + 2)
        he, hc = pend[ci % 3]
        he.wait()
        hc.wait()
        e_ref = rows_e[ci % 3]
        c_ref = rows_c[ci % 3]

        def _quad(gq, vec, ci=ci, e_ref=e_ref, c_ref=c_ref):
            def _row_acc(r):
                acc = e_ref[r, pl.ds(0, LANES)] * c_ref[r, pl.ds(0, LANES)]
                for k in range(1, EMB // LANES):
                    acc = acc + (e_ref[r, pl.ds(k * LANES, LANES)]
                                 * c_ref[r, pl.ds(k * LANES, LANES)])
                return acc

            def _pair(r):
                u = _row_acc(r)
                u = u + _perm(u, 1)
                v = _row_acc(r + 1)
                v = v + _perm(v, 1)
                return jnp.where(bitmask[0], u, v)

            r0 = gq * 4
            m01 = _pair(r0)
            m23 = _pair(r0 + 2)
            m01 = m01 + _perm(m01, 2)
            m23 = m23 + _perm(m23, 2)
            w = jnp.where(bitmask[1], m01, m23)
            w = w + _perm(w, 4)
            w = w + _perm(w, 8)
            q = gq & 3
            vec = jnp.where((lane >> 2) == q, w, vec)
            out_v[pl.ds(ci * CHUNK + (gq >> 2) * LANES, LANES)] = vec
            return vec

        lax.fori_loop(0, CHUNK // 4, _quad, jnp.zeros((LANES,), jnp.float32),
                      unroll=2)
    pltpu.sync_copy(out_v, out_hbm.at[pl.ds(base, PER_W)])


def _loss_body(ip_ref, lab_ref, out_ref):
    x = lab_ref[...] * ip_ref[...]
    out_ref[0, 0] = -(jnp.sum(jax.nn.log_sigmoid(x)) / jnp.float32(BATCH))


_loss = pl.pallas_call(
    _loss_body,
    out_shape=jax.ShapeDtypeStruct((1, 1), jnp.float32),
    out_specs=pl.BlockSpec(memory_space=pltpu.SMEM),
)


def kernel(u_i, u_j, label, embeddings, context_embedding):
    ui = u_i.astype(jnp.int32)
    uj = u_j.astype(jnp.int32)
    ip = _sc_dot(embeddings, context_embedding, ui, uj)
    out = _loss(ip.reshape(EMB, EMB), label.reshape(EMB, EMB))
    return out[0, 0]


# speedup vs baseline: 1.0852x; 1.0852x over previous
import functools

import jax
import jax.numpy as jnp
from jax import lax
from jax.experimental import pallas as pl
from jax.experimental.pallas import tpu as pltpu
from jax.experimental.pallas import tpu_sc as plsc

NODE = 100000
EMB = 128
BATCH = 16384
NC = 2
NS = 16
NW = NC * NS
PER_W = BATCH // NW
CHUNK = 64
N_CHUNK = PER_W // CHUNK
LANES = 16

_mesh = plsc.VectorSubcoreMesh(core_axis_name="c", subcore_axis_name="s")


@functools.partial(
    pl.kernel,
    mesh=_mesh,
    out_type=jax.ShapeDtypeStruct((BATCH,), jnp.float32),
    scratch_types=[
        pltpu.VMEM((PER_W,), jnp.int32),
        pltpu.VMEM((PER_W,), jnp.int32),
        pltpu.VMEM((CHUNK, EMB), jnp.float32),
        pltpu.VMEM((CHUNK, EMB), jnp.float32),
        pltpu.VMEM((CHUNK, EMB), jnp.float32),
        pltpu.VMEM((CHUNK, EMB), jnp.float32),
        pltpu.VMEM((CHUNK, EMB), jnp.float32),
        pltpu.VMEM((CHUNK, EMB), jnp.float32),
        pltpu.VMEM((CHUNK, EMB), jnp.float32),
        pltpu.VMEM((CHUNK, EMB), jnp.float32),
        pltpu.VMEM((PER_W,), jnp.float32),
        pltpu.SemaphoreType.DMA,
        pltpu.SemaphoreType.DMA,
        pltpu.SemaphoreType.DMA,
        pltpu.SemaphoreType.DMA,
        pltpu.SemaphoreType.DMA,
        pltpu.SemaphoreType.DMA,
        pltpu.SemaphoreType.DMA,
        pltpu.SemaphoreType.DMA,
    ],
)
def _sc_dot(emb_hbm, ctx_hbm, ui_hbm, uj_hbm, out_hbm,
            idx_i, idx_j,
            re0, re1, re2, re3, rc0, rc1, rc2, rc3, out_v,
            se0, se1, se2, se3, sc0, sc1, sc2, sc3):
    c = lax.axis_index("c")
    s = lax.axis_index("s")
    wid = s * NC + c
    base = pl.multiple_of(wid * PER_W, PER_W)
    lane = lax.iota(jnp.int32, LANES)
    bitmask = [((lane >> k) & 1) == 0 for k in range(4)]
    rows_e = (re0, re1, re2, re3)
    rows_c = (rc0, rc1, rc2, rc3)
    sem_e = (se0, se1, se2, se3)
    sem_c = (sc0, sc1, sc2, sc3)

    pltpu.sync_copy(ui_hbm.at[pl.ds(base, PER_W)], idx_i)
    pltpu.sync_copy(uj_hbm.at[pl.ds(base, PER_W)], idx_j)

    def _issue(ci):
        b = ci % 4
        sl = pl.ds(ci * CHUNK, CHUNK)
        he = pltpu.async_copy(emb_hbm.at[idx_i.at[sl]], rows_e[b], sem_e[b])
        hc = pltpu.async_copy(ctx_hbm.at[idx_j.at[sl]], rows_c[b], sem_c[b])
        return he, hc

    def _perm(x, m):
        return x.at[lane ^ m].get(mode="promise_in_bounds")

    pend = [None, None, None, None]
    pend[0] = _issue(0)
    pend[1] = _issue(1)
    pend[2] = _issue(2)
    for ci in range(N_CHUNK):
        if ci + 3 < N_CHUNK:
            pend[(ci + 3) % 4] = _issue(ci + 3)
        he, hc = pend[ci % 4]
        he.wait()
        hc.wait()
        e_ref = rows_e[ci % 4]
        c_ref = rows_c[ci % 4]

        def _quad(gq, vec, ci=ci, e_ref=e_ref, c_ref=c_ref):
            def _row_acc(r):
                acc = e_ref[r, pl.ds(0, LANES)] * c_ref[r, pl.ds(0, LANES)]
                for k in range(1, EMB // LANES):
                    acc = acc + (e_ref[r, pl.ds(k * LANES, LANES)]
                                 * c_ref[r, pl.ds(k * LANES, LANES)])
                return acc

            def _pair(r):
                u = _row_acc(r)
                u = u + _perm(u, 1)
                v = _row_acc(r + 1)
                v = v + _perm(v, 1)
                return jnp.where(bitmask[0], u, v)

            r0 = gq * 4
            m01 = _pair(r0)
            m23 = _pair(r0 + 2)
            m01 = m01 + _perm(m01, 2)
            m23 = m23 + _perm(m23, 2)
            w = jnp.where(bitmask[1], m01, m23)
            w = w + _perm(w, 4)
            w = w + _perm(w, 8)
            q = gq & 3
            vec = jnp.where((lane >> 2) == q, w, vec)
            out_v[pl.ds(ci * CHUNK + (gq >> 2) * LANES, LANES)] = vec
            return vec

        lax.fori_loop(0, CHUNK // 4, _quad, jnp.zeros((LANES,), jnp.float32),
                      unroll=2)
    pltpu.sync_copy(out_v, out_hbm.at[pl.ds(base, PER_W)])


def _loss_body(ip_ref, lab_ref, out_ref):
    x = lab_ref[...] * ip_ref[...]
    out_ref[0, 0] = -(jnp.sum(jax.nn.log_sigmoid(x)) / jnp.float32(BATCH))


_loss = pl.pallas_call(
    _loss_body,
    out_shape=jax.ShapeDtypeStruct((1, 1), jnp.float32),
    out_specs=pl.BlockSpec(memory_space=pltpu.SMEM),
)


def kernel(u_i, u_j, label, embeddings, context_embedding):
    ui = u_i.astype(jnp.int32)
    uj = u_j.astype(jnp.int32)
    ip = _sc_dot(embeddings, context_embedding, ui, uj)
    out = _loss(ip.reshape(EMB, EMB), label.reshape(EMB, EMB))
    return out[0, 0]


# speedup vs baseline: 1.0940x; 1.0082x over previous
import functools

import jax
import jax.numpy as jnp
from jax import lax
from jax.experimental import pallas as pl
from jax.experimental.pallas import tpu as pltpu
from jax.experimental.pallas import tpu_sc as plsc

NODE = 100000
EMB = 128
BATCH = 16384
NC = 2
NS = 16
NW = NC * NS
PER_W = BATCH // NW
CHUNK = 64
N_CHUNK = PER_W // CHUNK
LANES = 16

_mesh = plsc.VectorSubcoreMesh(core_axis_name="c", subcore_axis_name="s")


@functools.partial(
    pl.kernel,
    mesh=_mesh,
    out_type=jax.ShapeDtypeStruct((BATCH,), jnp.float32),
    scratch_types=[
        pltpu.VMEM((PER_W,), jnp.int32),
        pltpu.VMEM((PER_W,), jnp.int32),
        pltpu.VMEM((CHUNK, EMB), jnp.float32),
        pltpu.VMEM((CHUNK, EMB), jnp.float32),
        pltpu.VMEM((CHUNK, EMB), jnp.float32),
        pltpu.VMEM((CHUNK, EMB), jnp.float32),
        pltpu.VMEM((CHUNK, EMB), jnp.float32),
        pltpu.VMEM((CHUNK, EMB), jnp.float32),
        pltpu.VMEM((PER_W,), jnp.float32),
        pltpu.SemaphoreType.DMA,
        pltpu.SemaphoreType.DMA,
        pltpu.SemaphoreType.DMA,
        pltpu.SemaphoreType.DMA,
        pltpu.SemaphoreType.DMA,
        pltpu.SemaphoreType.DMA,
    ],
)
def _sc_dot(emb_hbm, ctx_hbm, ui_hbm, uj_hbm, out_hbm,
            idx_i, idx_j,
            re0, re1, re2, rc0, rc1, rc2, out_v,
            se0, se1, se2, sc0, sc1, sc2):
    c = lax.axis_index("c")
    s = lax.axis_index("s")
    wid = s * NC + c
    base = pl.multiple_of(wid * PER_W, PER_W)
    lane = lax.iota(jnp.int32, LANES)
    bitmask = [((lane >> k) & 1) == 0 for k in range(4)]
    rows_e = (re0, re1, re2)
    rows_c = (rc0, rc1, rc2)
    sem_e = (se0, se1, se2)
    sem_c = (sc0, sc1, sc2)

    pltpu.sync_copy(ui_hbm.at[pl.ds(base, PER_W)], idx_i)
    pltpu.sync_copy(uj_hbm.at[pl.ds(base, PER_W)], idx_j)

    def _issue(ci):
        b = ci % 3
        sl = pl.ds(ci * CHUNK, CHUNK)
        he = pltpu.async_copy(emb_hbm.at[idx_i.at[sl]], rows_e[b], sem_e[b])
        hc = pltpu.async_copy(ctx_hbm.at[idx_j.at[sl]], rows_c[b], sem_c[b])
        return he, hc

    def _perm(x, m):
        return x.at[lane ^ m].get(mode="promise_in_bounds")

    pend = [None, None, None]
    pend[0] = _issue(0)
    pend[1] = _issue(1)
    for ci in range(N_CHUNK):
        if ci + 2 < N_CHUNK:
            pend[(ci + 2) % 3] = _issue(ci + 2)
        he, hc = pend[ci % 3]
        he.wait()
        hc.wait()
        e_ref = rows_e[ci % 3]
        c_ref = rows_c[ci % 3]

        def _quad(gq, vec, ci=ci, e_ref=e_ref, c_ref=c_ref):
            def _row_acc(r):
                acc = e_ref[r, pl.ds(0, LANES)] * c_ref[r, pl.ds(0, LANES)]
                for k in range(1, EMB // LANES):
                    acc = acc + (e_ref[r, pl.ds(k * LANES, LANES)]
                                 * c_ref[r, pl.ds(k * LANES, LANES)])
                return acc

            def _pair(r):
                u = _row_acc(r)
                u = u + _perm(u, 1)
                v = _row_acc(r + 1)
                v = v + _perm(v, 1)
                return jnp.where(bitmask[0], u, v)

            r0 = gq * 4
            m01 = _pair(r0)
            m23 = _pair(r0 + 2)
            m01 = m01 + _perm(m01, 2)
            m23 = m23 + _perm(m23, 2)
            w = jnp.where(bitmask[1], m01, m23)
            w = w + _perm(w, 4)
            w = w + _perm(w, 8)
            q = gq & 3
            vec = jnp.where((lane >> 2) == q, w, vec)
            out_v[pl.ds(ci * CHUNK + (gq >> 2) * LANES, LANES)] = vec
            return vec

        lax.fori_loop(0, CHUNK // 4, _quad, jnp.zeros((LANES,), jnp.float32),
                      unroll=2)
    pltpu.sync_copy(out_v, out_hbm.at[pl.ds(base, PER_W)])


def _loss_body(ip_ref, lab_ref, out_ref):
    x = lab_ref[...] * ip_ref[...]
    out_ref[0, 0] = -(jnp.sum(jax.nn.log_sigmoid(x)) / jnp.float32(BATCH))


_loss = pl.pallas_call(
    _loss_body,
    out_shape=jax.ShapeDtypeStruct((1, 1), jnp.float32),
    out_specs=pl.BlockSpec(memory_space=pltpu.SMEM),
)


def kernel(u_i, u_j, label, embeddings, context_embedding):
    ui = u_i.astype(jnp.int32)
    uj = u_j.astype(jnp.int32)
    ip = _sc_dot(embeddings, context_embedding, ui, uj)
    out = _loss(ip.reshape(EMB, EMB), label.reshape(EMB, EMB))
    return out[0, 0]
